# Initial kernel scaffold; baseline (speedup 1.0000x reference)
import jax
import jax.numpy as jnp
from jax.experimental import pallas as pl


def kernel(x, chosen_idx, chosen_w, compute_mask, route_scale, W1, b1, W2, b2):
    raise NotImplementedError("write your pallas kernel here")


# speedup vs baseline: 1.1619x; 1.1619x over previous
import functools

import jax
import jax.numpy as jnp
from jax.experimental import pallas as pl
from jax.experimental.pallas import tpu as pltpu


def _moe_dense_body(x_ref, idx_ref, w_ref, cm_ref, scale_ref,
                    W1_ref, b1_ref, W2_ref, b2_ref, out_ref, util_ref, *, E):
    e = pl.program_id(1)
    xb = x_ref[...]
    idxb = idx_ref[...]
    wb = w_ref[...]
    cmb = cm_ref[...]
    scale = scale_ref[0, 0]

    wsum = jnp.clip(jnp.sum(wb, axis=1, keepdims=True), 1e-12, None)
    nw = wb / wsum
    nwm = nw * cmb
    coeff = jnp.sum(jnp.where(idxb == e, nwm, 0.0), axis=1, keepdims=True)

    h = jnp.maximum(jnp.dot(xb, W1_ref[0], preferred_element_type=jnp.float32)
                    + b1_ref[0], 0.0)
    z = jnp.dot(h, W2_ref[0], preferred_element_type=jnp.float32) + b2_ref[0]
    contrib = (scale * coeff) * z

    @pl.when(e == 0)
    def _():
        snw = jnp.sum(nw, axis=1, keepdims=True)
        out_ref[...] = (scale * snw) * xb + contrib
        TB = idxb.shape[0]
        eids = jax.lax.broadcasted_iota(jnp.int32, (TB, E), 1)
        acc = jnp.zeros((TB, E), jnp.float32)
        for k in range(idxb.shape[1]):
            hit = (idxb[:, k:k + 1] == eids) & (cmb[:, k:k + 1] > 0.0)
            acc = jnp.maximum(acc, hit.astype(jnp.float32))
        util_ref[...] = acc

    @pl.when(e > 0)
    def _():
        out_ref[...] += contrib


def kernel(x, chosen_idx, chosen_w, compute_mask, route_scale, W1, b1, W2, b2):
    B, N, T = x.shape
    K = chosen_idx.shape[-1]
    E, _, H = W1.shape
    BN = B * N
    x_flat = x.reshape(BN, T)
    idx = chosen_idx.reshape(BN, K).astype(jnp.int32)
    w = chosen_w.reshape(BN, K)
    cm = compute_mask.reshape(BN, K).astype(jnp.float32)
    scale = route_scale.reshape(1, 1).astype(jnp.float32)

    TB = 256
    NB = BN // TB

    out, util = pl.pallas_call(
        functools.partial(_moe_dense_body, E=E),
        grid=(NB, E),
        in_specs=[
            pl.BlockSpec((TB, T), lambda i, e: (i, 0)),
            pl.BlockSpec((TB, K), lambda i, e: (i, 0)),
            pl.BlockSpec((TB, K), lambda i, e: (i, 0)),
            pl.BlockSpec((TB, K), lambda i, e: (i, 0)),
            pl.BlockSpec(memory_space=pltpu.SMEM),
            pl.BlockSpec((1, T, H), lambda i, e: (e, 0, 0)),
            pl.BlockSpec((1, 1, H), lambda i, e: (e, 0, 0)),
            pl.BlockSpec((1, H, T), lambda i, e: (e, 0, 0)),
            pl.BlockSpec((1, 1, T), lambda i, e: (e, 0, 0)),
        ],
        out_specs=[
            pl.BlockSpec((TB, T), lambda i, e: (i, 0)),
            pl.BlockSpec((TB, E), lambda i, e: (i, 0)),
        ],
        out_shape=[
            jax.ShapeDtypeStruct((BN, T), jnp.float32),
            jax.ShapeDtypeStruct((BN, E), jnp.float32),
        ],
        compiler_params=pltpu.CompilerParams(
            dimension_semantics=("parallel", "arbitrary"),
        ),
    )(x_flat, idx, w, cm, scale, W1, b1.reshape(E, 1, H), W2, b2.reshape(E, 1, T))

    return out.reshape(B, N, T), util.reshape(B, N, E)


# speedup vs baseline: 1.7000x; 1.4631x over previous
import functools

import jax
import jax.numpy as jnp
from jax.experimental import pallas as pl
from jax.experimental.pallas import tpu as pltpu


def _moe_dense_body(x_ref, idx_ref, w_ref, cm_ref, scale_ref,
                    W1_ref, b1_ref, W2_ref, b2_ref, out_ref, util_ref, *, E):
    e = pl.program_id(1)
    xb = x_ref[...]
    idxb = idx_ref[...]
    wb = w_ref[...]
    cmb = cm_ref[...]
    scale = scale_ref[0, 0]

    wsum = jnp.clip(jnp.sum(wb, axis=1, keepdims=True), 1e-12, None)
    nw = wb / wsum
    nwm = nw * cmb
    coeff = jnp.sum(jnp.where(idxb == e, nwm, 0.0), axis=1, keepdims=True)

    h = jnp.maximum(jnp.dot(xb, W1_ref[0], preferred_element_type=jnp.float32)
                    + b1_ref[0], 0.0)
    z = jnp.dot(h.astype(W2_ref.dtype), W2_ref[0],
                preferred_element_type=jnp.float32) + b2_ref[0]
    contrib = (scale * coeff) * z

    @pl.when(e == 0)
    def _():
        snw = jnp.sum(nw, axis=1, keepdims=True)
        out_ref[...] = (scale * snw) * x_ref[...].astype(jnp.float32) + contrib
        TB = idxb.shape[0]
        eids = jax.lax.broadcasted_iota(jnp.int32, (TB, E), 1)
        acc = jnp.zeros((TB, E), jnp.float32)
        for k in range(idxb.shape[1]):
            hit = (idxb[:, k:k + 1] == eids) & (cmb[:, k:k + 1] > 0.0)
            acc = jnp.maximum(acc, hit.astype(jnp.float32))
        util_ref[...] = acc

    @pl.when(e > 0)
    def _():
        out_ref[...] += contrib


def kernel(x, chosen_idx, chosen_w, compute_mask, route_scale, W1, b1, W2, b2):
    B, N, T = x.shape
    K = chosen_idx.shape[-1]
    E, _, H = W1.shape
    BN = B * N
    x_flat = x.reshape(BN, T)
    idx = chosen_idx.reshape(BN, K).astype(jnp.int32)
    w = chosen_w.reshape(BN, K)
    cm = compute_mask.reshape(BN, K).astype(jnp.float32)
    scale = route_scale.reshape(1, 1).astype(jnp.float32)

    x16 = x_flat.astype(jnp.bfloat16)
    W1_16 = W1.astype(jnp.bfloat16)
    W2_16 = W2.astype(jnp.bfloat16)

    TB = 1024
    NB = BN // TB

    out, util = pl.pallas_call(
        functools.partial(_moe_dense_body, E=E),
        grid=(NB, E),
        in_specs=[
            pl.BlockSpec((TB, T), lambda i, e: (i, 0)),
            pl.BlockSpec((TB, K), lambda i, e: (i, 0)),
            pl.BlockSpec((TB, K), lambda i, e: (i, 0)),
            pl.BlockSpec((TB, K), lambda i, e: (i, 0)),
            pl.BlockSpec(memory_space=pltpu.SMEM),
            pl.BlockSpec((1, T, H), lambda i, e: (e, 0, 0)),
            pl.BlockSpec((1, 1, H), lambda i, e: (e, 0, 0)),
            pl.BlockSpec((1, H, T), lambda i, e: (e, 0, 0)),
            pl.BlockSpec((1, 1, T), lambda i, e: (e, 0, 0)),
        ],
        out_specs=[
            pl.BlockSpec((TB, T), lambda i, e: (i, 0)),
            pl.BlockSpec((TB, E), lambda i, e: (i, 0)),
        ],
        out_shape=[
            jax.ShapeDtypeStruct((BN, T), jnp.float32),
            jax.ShapeDtypeStruct((BN, E), jnp.float32),
        ],
        compiler_params=pltpu.CompilerParams(
            dimension_semantics=("parallel", "arbitrary"),
        ),
    )(x16, idx, w, cm, scale, W1_16, b1.reshape(E, 1, H), W2_16, b2.reshape(E, 1, T))

    return out.reshape(B, N, T), util.reshape(B, N, E)
